# Initial kernel scaffold; baseline (speedup 1.0000x reference)
import jax
import jax.numpy as jnp
from jax.experimental import pallas as pl


def kernel(x, mu):
    raise NotImplementedError("write your pallas kernel here")


# speedup vs baseline: 2.5529x; 2.5529x over previous
import jax
import jax.numpy as jnp
from jax.experimental import pallas as pl

BN = 256


def _body(x_ref, mu_ref, sim_ref, idx_ref):
    s = jax.lax.dot_general(
        x_ref[...], mu_ref[...],
        dimension_numbers=(((1,), (1,)), ((), ())),
        preferred_element_type=jnp.float32,
    )
    sim_ref[...] = s
    idx_ref[0, 0, :] = jnp.argmax(s, axis=1).astype(jnp.int32)


def kernel(x, mu):
    n, d = x.shape
    k = mu.shape[0]
    grid = (n // BN,)
    sim, idx3 = pl.pallas_call(
        _body,
        grid=grid,
        in_specs=[
            pl.BlockSpec((BN, d), lambda i: (i, 0)),
            pl.BlockSpec((k, d), lambda i: (0, 0)),
        ],
        out_specs=[
            pl.BlockSpec((BN, k), lambda i: (i, 0)),
            pl.BlockSpec((1, 1, BN), lambda i: (i, 0, 0)),
        ],
        out_shape=[
            jax.ShapeDtypeStruct((n, k), jnp.float32),
            jax.ShapeDtypeStruct((n // BN, 1, BN), jnp.int32),
        ],
    )(x, mu)
    return sim, idx3.reshape(n)
